# baseline (speedup 1.0000x reference)
import functools

import jax
import jax.numpy as jnp
from jax import lax
from jax.experimental import pallas as pl
from jax.experimental.pallas import tpu as pltpu
from jax.experimental.pallas import tpu_sc as plsc

N_NODES = 10000
N_EDGES = 320000
D = 128
DE = 16

NC = 2
NS = 16
L = 16

NP = 10240
EPT = N_EDGES // NS
CH = 4000
NCH = EPT // CH
U = 5
NODES_PER_TILE_RED = NP // NS
WIN = NP // (NC * NS)
GCH = 80
NGCH = WIN // GCH


def _scatter_max(wv, ids, pos):
  cur = plsc.load_gather(wv, [ids])
  plsc.store_scatter(wv, [ids], jnp.maximum(pos, cur))


def _sc_a_body(sid_hbm, did_hbm, winner_out,
               sid_st, did_st, wv0, wv1, wv2, wv3, wv4,
               shared_w, comb_sh, red_buf, red_out):
  c = lax.axis_index("c")
  s = lax.axis_index("s")
  lane = lax.iota(jnp.int32, L)
  ways = (wv0, wv1, wv2, wv3, wv4)

  def init_body(i, _):
    for wv in ways:
      wv[pl.ds(i * L, L)] = jnp.full((L,), -1, jnp.int32)
    return 0

  lax.fori_loop(0, NP // L, init_body, 0)

  ebase = s * EPT

  def chunk_body(ci, _):
    base = ebase + ci * CH
    pltpu.sync_copy(sid_hbm.at[pl.ds(base, CH)], sid_st)
    pltpu.sync_copy(did_hbm.at[pl.ds(base, CH)], did_st)

    def grp_body(g, _):
      off0 = g * (L * U)
      for u in range(U):
        off = off0 + u * L
        wv = ways[u]
        eidx = base + off + lane
        dpos = 2 * eidx
        dids = did_st[pl.ds(off, L)]
        sids = sid_st[pl.ds(off, L)]
        _scatter_max(wv, dids, dpos)
        _scatter_max(wv, sids, dpos + 1)
      return 0

    lax.fori_loop(0, CH // (L * U), grp_body, 0)
    return 0

  lax.fori_loop(0, NCH, chunk_body, 0)

  def way_body(i, _):
    m = ways[0][pl.ds(i * L, L)]
    for wv in ways[1:]:
      m = jnp.maximum(m, wv[pl.ds(i * L, L)])
    wv0[pl.ds(i * L, L)] = m
    return 0

  lax.fori_loop(0, NP // L, way_body, 0)

  pltpu.sync_copy(wv0, shared_w.at[pl.ds(s * NP, NP)])
  plsc.subcore_barrier()

  for half in range(NODES_PER_TILE_RED // WIN):
    nbase = s * NODES_PER_TILE_RED + half * WIN
    for j in range(NS):
      pltpu.sync_copy(shared_w.at[pl.ds(j * NP + nbase, WIN)],
                      red_buf.at[pl.ds(j * WIN, WIN)])

    def red_body(v, _):
      m = red_buf[pl.ds(v * L, L)]
      for j in range(1, NS):
        m = jnp.maximum(m, red_buf[pl.ds(j * WIN + v * L, L)])
      red_out[pl.ds(v * L, L)] = m
      return 0

    lax.fori_loop(0, WIN // L, red_body, 0)
    pltpu.sync_copy(red_out, comb_sh.at[pl.ds(nbase, WIN)])

  plsc.subcore_barrier()

  wbase = c * (NS * WIN) + s * WIN
  pltpu.sync_copy(comb_sh.at[pl.ds(wbase, WIN)], red_out)
  pltpu.sync_copy(red_out, winner_out.at[pl.ds(wbase, WIN)])


_sc_scatter_max = functools.partial(
    pl.kernel,
    out_type=jax.ShapeDtypeStruct((NP,), jnp.int32),
    mesh=plsc.VectorSubcoreMesh(
        core_axis_name="c", subcore_axis_name="s",
        num_cores=NC, num_subcores=NS),
    compiler_params=pltpu.CompilerParams(
        needs_layout_passes=False, use_tc_tiling_on_sc=False),
    scratch_types=[
        pltpu.VMEM((CH,), jnp.int32),
        pltpu.VMEM((CH,), jnp.int32),
        pltpu.VMEM((NP,), jnp.int32),
        pltpu.VMEM((NP,), jnp.int32),
        pltpu.VMEM((NP,), jnp.int32),
        pltpu.VMEM((NP,), jnp.int32),
        pltpu.VMEM((NP,), jnp.int32),
        pltpu.VMEM_SHARED((NS * NP,), jnp.int32),
        pltpu.VMEM_SHARED((NP,), jnp.int32),
        pltpu.VMEM((NS * WIN,), jnp.int32),
        pltpu.VMEM((WIN,), jnp.int32),
    ],
)(_sc_a_body)


def _sc_b_body(winner_hbm, emb_hbm, sid_hbm, did_hbm, ef1_hbm, tf1_hbm,
               semb_out, demb_out, eft_out, tft_out,
               win_w, e_idx, feat_idx, sid_w, did_w,
               eft_v, tft_v, semb_v, demb_v, sem, sem2):
  c = lax.axis_index("c")
  s = lax.axis_index("s")

  wbase = c * (NS * WIN) + s * WIN
  pltpu.sync_copy(winner_hbm.at[pl.ds(wbase, WIN)], win_w)

  for v in range(WIN // L):
    w = win_w[pl.ds(v * L, L)]
    e = jnp.maximum(w, 0) >> 1
    e_idx[v // (GCH // L), pl.ds((v % (GCH // L)) * L, L)] = e
    for f in range(DE):
      feat_idx[pl.ds(f * WIN + v * L, L)] = e + f * N_EDGES

  id_descs = []
  for j in range(NGCH):
    id_descs.append(
        pltpu.async_copy(sid_hbm.at[e_idx.at[j]], sid_w.at[j], sem2))
    id_descs.append(
        pltpu.async_copy(did_hbm.at[e_idx.at[j]], did_w.at[j], sem2))

  descs = []
  for f in range(DE):
    for j in range(NGCH):
      idx = feat_idx.at[pl.ds(f * WIN + j * GCH, GCH)]
      descs.append(pltpu.async_copy(
          ef1_hbm.at[idx], eft_v.at[pl.ds(f * WIN + j * GCH, GCH)], sem))
      descs.append(pltpu.async_copy(
          tf1_hbm.at[idx], tft_v.at[pl.ds(f * WIN + j * GCH, GCH)], sem))

  for d_ in id_descs:
    d_.wait()
  for j in range(NGCH):
    descs.append(pltpu.async_copy(
        emb_hbm.at[sid_w.at[j]], semb_v.at[pl.ds(j * GCH, GCH)], sem))
    descs.append(pltpu.async_copy(
        emb_hbm.at[did_w.at[j]], demb_v.at[pl.ds(j * GCH, GCH)], sem))
  for d_ in descs:
    d_.wait()

  pltpu.sync_copy(semb_v, semb_out.at[pl.ds(wbase, WIN)])
  pltpu.sync_copy(demb_v, demb_out.at[pl.ds(wbase, WIN)])
  for f in range(DE):
    pltpu.sync_copy(eft_v.at[pl.ds(f * WIN, WIN)],
                    eft_out.at[f, pl.ds(wbase, WIN)])
    pltpu.sync_copy(tft_v.at[pl.ds(f * WIN, WIN)],
                    tft_out.at[f, pl.ds(wbase, WIN)])


_sc_gather = functools.partial(
    pl.kernel,
    out_type=(
        jax.ShapeDtypeStruct((NP, D), jnp.float32),
        jax.ShapeDtypeStruct((NP, D), jnp.float32),
        jax.ShapeDtypeStruct((DE, NP), jnp.float32),
        jax.ShapeDtypeStruct((DE, NP), jnp.float32),
    ),
    mesh=plsc.VectorSubcoreMesh(
        core_axis_name="c", subcore_axis_name="s",
        num_cores=NC, num_subcores=NS),
    compiler_params=pltpu.CompilerParams(
        needs_layout_passes=False, use_tc_tiling_on_sc=False),
    scratch_types=[
        pltpu.VMEM((WIN,), jnp.int32),
        pltpu.VMEM((NGCH, GCH), jnp.int32),
        pltpu.VMEM((DE * WIN,), jnp.int32),
        pltpu.VMEM((NGCH, GCH), jnp.int32),
        pltpu.VMEM((NGCH, GCH), jnp.int32),
        pltpu.VMEM((DE * WIN,), jnp.float32),
        pltpu.VMEM((DE * WIN,), jnp.float32),
        pltpu.VMEM((WIN, D), jnp.float32),
        pltpu.VMEM((WIN, D), jnp.float32),
        pltpu.SemaphoreType.DMA,
        pltpu.SemaphoreType.DMA,
    ],
)(_sc_b_body)


BLK = 1024
GRID = NP // BLK


def _tc_body(win_ref, semb_ref, demb_ref, eft_ref, tft_ref, emb_ref,
             w1a, w1b, w1c, w1d, b1, w2, b2, u1a, u1b, ub1, u2, ub2,
             out_ref):
  dot = functools.partial(jnp.dot, preferred_element_type=jnp.float32)
  dot_t = functools.partial(
      lax.dot_general, dimension_numbers=(((0,), (0,)), ((), ())),
      preferred_element_type=jnp.float32)
  xs = semb_ref[...]
  xd = demb_ref[...]
  emb = emb_ref[...]
  h = (dot(xs, w1a[...]) + dot(xd, w1b[...]) + dot_t(eft_ref[...], w1c[...]) +
       dot_t(tft_ref[...], w1d[...]) + b1[...])
  h = jnp.maximum(h, 0.0)
  msg = dot(h, w2[...]) + b2[...]
  h2 = jnp.maximum(dot(emb, u1a[...]) + dot(msg, u1b[...]) + ub1[...], 0.0)
  upd = dot(h2, u2[...]) + ub2[...]
  valid = win_ref[...] >= 0
  nz = jnp.any(upd != 0.0, axis=1, keepdims=True)
  keep = valid & nz
  out_ref[...] = jnp.where(keep, upd, emb)


def _full2(i):
  return (0, 0)


_tc_mlp = pl.pallas_call(
    _tc_body,
    grid=(GRID,),
    in_specs=[
        pl.BlockSpec((BLK, 1), lambda i: (i, 0)),
        pl.BlockSpec((BLK, D), lambda i: (i, 0)),
        pl.BlockSpec((BLK, D), lambda i: (i, 0)),
        pl.BlockSpec((DE, BLK), lambda i: (0, i)),
        pl.BlockSpec((DE, BLK), lambda i: (0, i)),
        pl.BlockSpec((BLK, D), lambda i: (i, 0)),
        pl.BlockSpec((D, D), _full2),
        pl.BlockSpec((D, D), _full2),
        pl.BlockSpec((DE, D), _full2),
        pl.BlockSpec((DE, D), _full2),
        pl.BlockSpec((1, D), _full2),
        pl.BlockSpec((D, D), _full2),
        pl.BlockSpec((1, D), _full2),
        pl.BlockSpec((D, D), _full2),
        pl.BlockSpec((D, D), _full2),
        pl.BlockSpec((1, D), _full2),
        pl.BlockSpec((D, D), _full2),
        pl.BlockSpec((1, D), _full2),
    ],
    out_specs=pl.BlockSpec((BLK, D), lambda i: (i, 0)),
    out_shape=jax.ShapeDtypeStruct((NP, D), jnp.float32),
)


def kernel(node_embeddings, src_node_ids, dst_node_ids, edge_features,
           time_features, W1, b1, W2, b2, U1, ub1, U2, ub2):
  sid = src_node_ids.astype(jnp.int32)
  did = dst_node_ids.astype(jnp.int32)
  ef1 = edge_features.T.reshape(-1)
  tf1 = time_features.T.reshape(-1)

  winner = _sc_scatter_max(sid, did)
  semb, demb, eft, tft = _sc_gather(
      winner, node_embeddings, sid, did, ef1, tf1)

  embp = jnp.pad(node_embeddings, ((0, NP - N_NODES), (0, 0)))
  out = _tc_mlp(
      winner.reshape(NP, 1), semb, demb, eft, tft, embp,
      W1[:D], W1[D:2 * D], W1[2 * D:2 * D + DE], W1[2 * D + DE:],
      b1.reshape(1, D), W2, b2.reshape(1, D),
      U1[:D], U1[D:], ub1.reshape(1, D), U2, ub2.reshape(1, D))
  return out[:N_NODES]


# speedup vs baseline: 1.0626x; 1.0626x over previous
import functools

import jax
import jax.numpy as jnp
from jax import lax
from jax.experimental import pallas as pl
from jax.experimental.pallas import tpu as pltpu
from jax.experimental.pallas import tpu_sc as plsc

N_NODES = 10000
N_EDGES = 320000
D = 128
DE = 16

NC = 2
NS = 16
L = 16

NP = 10240
EPS = N_EDGES // NC
EPT = EPS // NS
CH = 2000
NCH = EPT // CH
U = 5
NODES_PER_TILE_RED = NP // NS
WIN = NP // (NC * NS)
GCH = 80
NGCH = WIN // GCH


def _scatter_max(wv, ids, pos):
  cur = plsc.load_gather(wv, [ids])
  plsc.store_scatter(wv, [ids], jnp.maximum(pos, cur))


def _sc_a_body(sid_hbm, did_hbm, winner_out,
               sid_st, did_st, wv0, wv1, wv2, wv3, wv4,
               shared_w, red_buf, red_out):
  c = lax.axis_index("c")
  s = lax.axis_index("s")
  lane = lax.iota(jnp.int32, L)
  ways = (wv0, wv1, wv2, wv3, wv4)

  def init_body(i, _):
    for wv in ways:
      wv[pl.ds(i * L, L)] = jnp.full((L,), -1, jnp.int32)
    return 0

  lax.fori_loop(0, NP // L, init_body, 0)

  ebase = c * EPS + s * EPT

  def chunk_body(ci, _):
    base = ebase + ci * CH
    pltpu.sync_copy(sid_hbm.at[pl.ds(base, CH)], sid_st)
    pltpu.sync_copy(did_hbm.at[pl.ds(base, CH)], did_st)

    def grp_body(g, _):
      off0 = g * (L * U)
      for u in range(U):
        off = off0 + u * L
        wv = ways[u]
        eidx = base + off + lane
        dpos = 2 * eidx
        dids = did_st[pl.ds(off, L)]
        sids = sid_st[pl.ds(off, L)]
        _scatter_max(wv, dids, dpos)
        _scatter_max(wv, sids, dpos + 1)
      return 0

    lax.fori_loop(0, CH // (L * U), grp_body, 0)
    return 0

  lax.fori_loop(0, NCH, chunk_body, 0)

  def way_body(i, _):
    m = ways[0][pl.ds(i * L, L)]
    for wv in ways[1:]:
      m = jnp.maximum(m, wv[pl.ds(i * L, L)])
    wv0[pl.ds(i * L, L)] = m
    return 0

  lax.fori_loop(0, NP // L, way_body, 0)

  pltpu.sync_copy(wv0, shared_w.at[pl.ds(s * NP, NP)])
  plsc.subcore_barrier()

  for half in range(NODES_PER_TILE_RED // WIN):
    nbase = s * NODES_PER_TILE_RED + half * WIN
    for j in range(NS):
      pltpu.sync_copy(shared_w.at[pl.ds(j * NP + nbase, WIN)],
                      red_buf.at[pl.ds(j * WIN, WIN)])

    def red_body(v, _):
      m = red_buf[pl.ds(v * L, L)]
      for j in range(1, NS):
        m = jnp.maximum(m, red_buf[pl.ds(j * WIN + v * L, L)])
      red_out[pl.ds(v * L, L)] = m
      return 0

    lax.fori_loop(0, WIN // L, red_body, 0)
    pltpu.sync_copy(red_out, winner_out.at[pl.ds(c * NP + nbase, WIN)])


_sc_scatter_max = functools.partial(
    pl.kernel,
    out_type=jax.ShapeDtypeStruct((NC * NP,), jnp.int32),
    mesh=plsc.VectorSubcoreMesh(
        core_axis_name="c", subcore_axis_name="s",
        num_cores=NC, num_subcores=NS),
    compiler_params=pltpu.CompilerParams(
        needs_layout_passes=False, use_tc_tiling_on_sc=False),
    scratch_types=[
        pltpu.VMEM((CH,), jnp.int32),
        pltpu.VMEM((CH,), jnp.int32),
        pltpu.VMEM((NP,), jnp.int32),
        pltpu.VMEM((NP,), jnp.int32),
        pltpu.VMEM((NP,), jnp.int32),
        pltpu.VMEM((NP,), jnp.int32),
        pltpu.VMEM((NP,), jnp.int32),
        pltpu.VMEM_SHARED((NS * NP,), jnp.int32),
        pltpu.VMEM((NS * WIN,), jnp.int32),
        pltpu.VMEM((WIN,), jnp.int32),
    ],
)(_sc_a_body)


def _sc_b_body(winner_hbm, emb_hbm, sid_hbm, did_hbm, ef1_hbm, tf1_hbm,
               winc_out, semb_out, demb_out, eft_out, tft_out,
               win_w, win_w2, e_idx, feat_idx, sid_w, did_w,
               eft_v, tft_v, semb_v, demb_v, sem, sem2):
  c = lax.axis_index("c")
  s = lax.axis_index("s")

  wbase = c * (NS * WIN) + s * WIN
  pltpu.sync_copy(winner_hbm.at[pl.ds(wbase, WIN)], win_w)
  pltpu.sync_copy(winner_hbm.at[pl.ds(NP + wbase, WIN)], win_w2)

  for v in range(WIN // L):
    w = jnp.maximum(win_w[pl.ds(v * L, L)], win_w2[pl.ds(v * L, L)])
    win_w[pl.ds(v * L, L)] = w
    e = jnp.maximum(w, 0) >> 1
    e_idx[v // (GCH // L), pl.ds((v % (GCH // L)) * L, L)] = e
    for f in range(DE):
      feat_idx[pl.ds(f * WIN + v * L, L)] = e + f * N_EDGES

  id_descs = []
  for j in range(NGCH):
    id_descs.append(
        pltpu.async_copy(sid_hbm.at[e_idx.at[j]], sid_w.at[j], sem2))
    id_descs.append(
        pltpu.async_copy(did_hbm.at[e_idx.at[j]], did_w.at[j], sem2))

  descs = []
  for f in range(DE):
    for j in range(NGCH):
      idx = feat_idx.at[pl.ds(f * WIN + j * GCH, GCH)]
      descs.append(pltpu.async_copy(
          ef1_hbm.at[idx], eft_v.at[pl.ds(f * WIN + j * GCH, GCH)], sem))
      descs.append(pltpu.async_copy(
          tf1_hbm.at[idx], tft_v.at[pl.ds(f * WIN + j * GCH, GCH)], sem))

  for d_ in id_descs:
    d_.wait()
  for j in range(NGCH):
    descs.append(pltpu.async_copy(
        emb_hbm.at[sid_w.at[j]], semb_v.at[pl.ds(j * GCH, GCH)], sem))
    descs.append(pltpu.async_copy(
        emb_hbm.at[did_w.at[j]], demb_v.at[pl.ds(j * GCH, GCH)], sem))
  for d_ in descs:
    d_.wait()

  pltpu.sync_copy(win_w, winc_out.at[pl.ds(wbase, WIN)])
  pltpu.sync_copy(semb_v, semb_out.at[pl.ds(wbase, WIN)])
  pltpu.sync_copy(demb_v, demb_out.at[pl.ds(wbase, WIN)])
  for f in range(DE):
    pltpu.sync_copy(eft_v.at[pl.ds(f * WIN, WIN)],
                    eft_out.at[f, pl.ds(wbase, WIN)])
    pltpu.sync_copy(tft_v.at[pl.ds(f * WIN, WIN)],
                    tft_out.at[f, pl.ds(wbase, WIN)])


_sc_gather = functools.partial(
    pl.kernel,
    out_type=(
        jax.ShapeDtypeStruct((NP,), jnp.int32),
        jax.ShapeDtypeStruct((NP, D), jnp.float32),
        jax.ShapeDtypeStruct((NP, D), jnp.float32),
        jax.ShapeDtypeStruct((DE, NP), jnp.float32),
        jax.ShapeDtypeStruct((DE, NP), jnp.float32),
    ),
    mesh=plsc.VectorSubcoreMesh(
        core_axis_name="c", subcore_axis_name="s",
        num_cores=NC, num_subcores=NS),
    compiler_params=pltpu.CompilerParams(
        needs_layout_passes=False, use_tc_tiling_on_sc=False),
    scratch_types=[
        pltpu.VMEM((WIN,), jnp.int32),
        pltpu.VMEM((WIN,), jnp.int32),
        pltpu.VMEM((NGCH, GCH), jnp.int32),
        pltpu.VMEM((DE * WIN,), jnp.int32),
        pltpu.VMEM((NGCH, GCH), jnp.int32),
        pltpu.VMEM((NGCH, GCH), jnp.int32),
        pltpu.VMEM((DE * WIN,), jnp.float32),
        pltpu.VMEM((DE * WIN,), jnp.float32),
        pltpu.VMEM((WIN, D), jnp.float32),
        pltpu.VMEM((WIN, D), jnp.float32),
        pltpu.SemaphoreType.DMA,
        pltpu.SemaphoreType.DMA,
    ],
)(_sc_b_body)


BLK = 1024
GRID = NP // BLK


def _tc_body(win_ref, semb_ref, demb_ref, eft_ref, tft_ref, emb_ref,
             w1a, w1b, w1c, w1d, b1, w2, b2, u1a, u1b, ub1, u2, ub2,
             out_ref):
  dot = functools.partial(jnp.dot, preferred_element_type=jnp.float32)
  dot_t = functools.partial(
      lax.dot_general, dimension_numbers=(((0,), (0,)), ((), ())),
      preferred_element_type=jnp.float32)
  xs = semb_ref[...]
  xd = demb_ref[...]
  emb = emb_ref[...]
  h = (dot(xs, w1a[...]) + dot(xd, w1b[...]) + dot_t(eft_ref[...], w1c[...]) +
       dot_t(tft_ref[...], w1d[...]) + b1[...])
  h = jnp.maximum(h, 0.0)
  msg = dot(h, w2[...]) + b2[...]
  h2 = jnp.maximum(dot(emb, u1a[...]) + dot(msg, u1b[...]) + ub1[...], 0.0)
  upd = dot(h2, u2[...]) + ub2[...]
  valid = win_ref[...] >= 0
  nz = jnp.any(upd != 0.0, axis=1, keepdims=True)
  keep = valid & nz
  out_ref[...] = jnp.where(keep, upd, emb)


def _full2(i):
  return (0, 0)


_tc_mlp = pl.pallas_call(
    _tc_body,
    grid=(GRID,),
    in_specs=[
        pl.BlockSpec((BLK, 1), lambda i: (i, 0)),
        pl.BlockSpec((BLK, D), lambda i: (i, 0)),
        pl.BlockSpec((BLK, D), lambda i: (i, 0)),
        pl.BlockSpec((DE, BLK), lambda i: (0, i)),
        pl.BlockSpec((DE, BLK), lambda i: (0, i)),
        pl.BlockSpec((BLK, D), lambda i: (i, 0)),
        pl.BlockSpec((D, D), _full2),
        pl.BlockSpec((D, D), _full2),
        pl.BlockSpec((DE, D), _full2),
        pl.BlockSpec((DE, D), _full2),
        pl.BlockSpec((1, D), _full2),
        pl.BlockSpec((D, D), _full2),
        pl.BlockSpec((1, D), _full2),
        pl.BlockSpec((D, D), _full2),
        pl.BlockSpec((D, D), _full2),
        pl.BlockSpec((1, D), _full2),
        pl.BlockSpec((D, D), _full2),
        pl.BlockSpec((1, D), _full2),
    ],
    out_specs=pl.BlockSpec((BLK, D), lambda i: (i, 0)),
    out_shape=jax.ShapeDtypeStruct((NP, D), jnp.float32),
)


def kernel(node_embeddings, src_node_ids, dst_node_ids, edge_features,
           time_features, W1, b1, W2, b2, U1, ub1, U2, ub2):
  sid = src_node_ids.astype(jnp.int32)
  did = dst_node_ids.astype(jnp.int32)
  ef1 = edge_features.T.reshape(-1)
  tf1 = time_features.T.reshape(-1)

  partials = _sc_scatter_max(sid, did)
  winner, semb, demb, eft, tft = _sc_gather(
      partials, node_embeddings, sid, did, ef1, tf1)

  embp = jnp.pad(node_embeddings, ((0, NP - N_NODES), (0, 0)))
  out = _tc_mlp(
      winner.reshape(NP, 1), semb, demb, eft, tft, embp,
      W1[:D], W1[D:2 * D], W1[2 * D:2 * D + DE], W1[2 * D + DE:],
      b1.reshape(1, D), W2, b2.reshape(1, D),
      U1[:D], U1[D:], ub1.reshape(1, D), U2, ub2.reshape(1, D))
  return out[:N_NODES]


# speedup vs baseline: 1.0969x; 1.0322x over previous
import functools

import jax
import jax.numpy as jnp
from jax import lax
from jax.experimental import pallas as pl
from jax.experimental.pallas import tpu as pltpu
from jax.experimental.pallas import tpu_sc as plsc

N_NODES = 10000
N_EDGES = 320000
D = 128
DE = 16

NC = 2
NS = 16
L = 16

NP = 10240
EPS = N_EDGES // NC
EPT = EPS // NS
CH = 2000
NCH = EPT // CH
U = 5
NODES_PER_TILE_RED = NP // NS
WIN = NP // (NC * NS)
GCH = 80
NGCH = WIN // GCH


def _scatter_max(wv, ids, pos):
  cur = plsc.load_gather(wv, [ids])
  plsc.store_scatter(wv, [ids], jnp.maximum(pos, cur))


def _sc_a_body(sid_hbm, did_hbm, winner_out,
               sid_st, did_st, wv0, wv1, wv2, wv3, wv4,
               shared_w, red_buf, red_out):
  c = lax.axis_index("c")
  s = lax.axis_index("s")
  lane = lax.iota(jnp.int32, L)
  ways = (wv0, wv1, wv2, wv3, wv4)

  def init_body(i, _):
    for wv in ways:
      wv[pl.ds(i * L, L)] = jnp.full((L,), -1, jnp.int32)
    return 0

  lax.fori_loop(0, NP // L, init_body, 0)

  ebase = c * EPS + s * EPT

  def chunk_body(ci, _):
    base = ebase + ci * CH
    pltpu.sync_copy(sid_hbm.at[pl.ds(base, CH)], sid_st)
    pltpu.sync_copy(did_hbm.at[pl.ds(base, CH)], did_st)

    def grp_body(g, _):
      off0 = g * (L * U)
      for u in range(U):
        off = off0 + u * L
        wv = ways[u]
        eidx = base + off + lane
        dpos = 2 * eidx
        dids = did_st[pl.ds(off, L)]
        sids = sid_st[pl.ds(off, L)]
        _scatter_max(wv, dids, dpos)
        _scatter_max(wv, sids, dpos + 1)
      return 0

    lax.fori_loop(0, CH // (L * U), grp_body, 0)
    return 0

  lax.fori_loop(0, NCH, chunk_body, 0)

  def way_body(i, _):
    m = ways[0][pl.ds(i * L, L)]
    for wv in ways[1:]:
      m = jnp.maximum(m, wv[pl.ds(i * L, L)])
    wv0[pl.ds(i * L, L)] = m
    return 0

  lax.fori_loop(0, NP // L, way_body, 0)

  pltpu.sync_copy(wv0, shared_w.at[pl.ds(s * NP, NP)])
  plsc.subcore_barrier()

  for half in range(NODES_PER_TILE_RED // WIN):
    nbase = s * NODES_PER_TILE_RED + half * WIN
    for j in range(NS):
      pltpu.sync_copy(shared_w.at[pl.ds(j * NP + nbase, WIN)],
                      red_buf.at[pl.ds(j * WIN, WIN)])

    def red_body(v, _):
      m = red_buf[pl.ds(v * L, L)]
      for j in range(1, NS):
        m = jnp.maximum(m, red_buf[pl.ds(j * WIN + v * L, L)])
      red_out[pl.ds(v * L, L)] = m
      return 0

    lax.fori_loop(0, WIN // L, red_body, 0)
    pltpu.sync_copy(red_out, winner_out.at[pl.ds(c * NP + nbase, WIN)])


_sc_scatter_max = functools.partial(
    pl.kernel,
    out_type=jax.ShapeDtypeStruct((NC * NP,), jnp.int32),
    mesh=plsc.VectorSubcoreMesh(
        core_axis_name="c", subcore_axis_name="s",
        num_cores=NC, num_subcores=NS),
    compiler_params=pltpu.CompilerParams(
        needs_layout_passes=False, use_tc_tiling_on_sc=False),
    scratch_types=[
        pltpu.VMEM((CH,), jnp.int32),
        pltpu.VMEM((CH,), jnp.int32),
        pltpu.VMEM((NP,), jnp.int32),
        pltpu.VMEM((NP,), jnp.int32),
        pltpu.VMEM((NP,), jnp.int32),
        pltpu.VMEM((NP,), jnp.int32),
        pltpu.VMEM((NP,), jnp.int32),
        pltpu.VMEM_SHARED((NS * NP,), jnp.int32),
        pltpu.VMEM((NS * WIN,), jnp.int32),
        pltpu.VMEM((WIN,), jnp.int32),
    ],
)(_sc_a_body)


def _sc_b_body(winner_hbm, emb_hbm, sid_hbm, did_hbm, ef1_hbm, tf1_hbm,
               winc_out, oemb_out, eft_out, tft_out,
               win_w, win_w2, e_idx, feat_idx, sid_w, did_w,
               eft_v, tft_v, oemb_v, sem, sem2):
  c = lax.axis_index("c")
  s = lax.axis_index("s")

  wbase = c * (NS * WIN) + s * WIN
  pltpu.sync_copy(winner_hbm.at[pl.ds(wbase, WIN)], win_w)
  pltpu.sync_copy(winner_hbm.at[pl.ds(NP + wbase, WIN)], win_w2)

  for v in range(WIN // L):
    w = jnp.maximum(win_w[pl.ds(v * L, L)], win_w2[pl.ds(v * L, L)])
    win_w[pl.ds(v * L, L)] = w
    e = jnp.maximum(w, 0) >> 1
    e_idx[v // (GCH // L), pl.ds((v % (GCH // L)) * L, L)] = e
    for f in range(DE):
      feat_idx[pl.ds(f * WIN + v * L, L)] = e + f * N_EDGES

  id_descs = []
  for j in range(NGCH):
    id_descs.append(
        pltpu.async_copy(sid_hbm.at[e_idx.at[j]], sid_w.at[j], sem2))
    id_descs.append(
        pltpu.async_copy(did_hbm.at[e_idx.at[j]], did_w.at[j], sem2))

  descs = []
  for f in range(DE):
    for j in range(NGCH):
      idx = feat_idx.at[pl.ds(f * WIN + j * GCH, GCH)]
      descs.append(pltpu.async_copy(
          ef1_hbm.at[idx], eft_v.at[pl.ds(f * WIN + j * GCH, GCH)], sem))
      descs.append(pltpu.async_copy(
          tf1_hbm.at[idx], tft_v.at[pl.ds(f * WIN + j * GCH, GCH)], sem))

  for d_ in id_descs:
    d_.wait()
  for v in range(WIN // L):
    j, k = v // (GCH // L), (v % (GCH // L)) * L
    role = win_w[pl.ds(v * L, L)] & 1
    oth = jnp.where(role == 0, sid_w[j, pl.ds(k, L)], did_w[j, pl.ds(k, L)])
    sid_w[j, pl.ds(k, L)] = oth
  for j in range(NGCH):
    descs.append(pltpu.async_copy(
        emb_hbm.at[sid_w.at[j]], oemb_v.at[pl.ds(j * GCH, GCH)], sem))
  for d_ in descs:
    d_.wait()

  pltpu.sync_copy(win_w, winc_out.at[pl.ds(wbase, WIN)])
  pltpu.sync_copy(oemb_v, oemb_out.at[pl.ds(wbase, WIN)])
  for f in range(DE):
    pltpu.sync_copy(eft_v.at[pl.ds(f * WIN, WIN)],
                    eft_out.at[f, pl.ds(wbase, WIN)])
    pltpu.sync_copy(tft_v.at[pl.ds(f * WIN, WIN)],
                    tft_out.at[f, pl.ds(wbase, WIN)])


_sc_gather = functools.partial(
    pl.kernel,
    out_type=(
        jax.ShapeDtypeStruct((NP,), jnp.int32),
        jax.ShapeDtypeStruct((NP, D), jnp.float32),
        jax.ShapeDtypeStruct((DE, NP), jnp.float32),
        jax.ShapeDtypeStruct((DE, NP), jnp.float32),
    ),
    mesh=plsc.VectorSubcoreMesh(
        core_axis_name="c", subcore_axis_name="s",
        num_cores=NC, num_subcores=NS),
    compiler_params=pltpu.CompilerParams(
        needs_layout_passes=False, use_tc_tiling_on_sc=False),
    scratch_types=[
        pltpu.VMEM((WIN,), jnp.int32),
        pltpu.VMEM((WIN,), jnp.int32),
        pltpu.VMEM((NGCH, GCH), jnp.int32),
        pltpu.VMEM((DE * WIN,), jnp.int32),
        pltpu.VMEM((NGCH, GCH), jnp.int32),
        pltpu.VMEM((NGCH, GCH), jnp.int32),
        pltpu.VMEM((DE * WIN,), jnp.float32),
        pltpu.VMEM((DE * WIN,), jnp.float32),
        pltpu.VMEM((WIN, D), jnp.float32),
        pltpu.SemaphoreType.DMA,
        pltpu.SemaphoreType.DMA,
    ],
)(_sc_b_body)


BLK = 1024
GRID = NP // BLK


def _tc_body(win_ref, oemb_ref, eft_ref, tft_ref, emb_ref,
             w1a, w1b, w1ab, w1c, w1d, b1, w2, b2, u1a, u1b, ub1, u2, ub2,
             out_ref):
  dot = functools.partial(jnp.dot, preferred_element_type=jnp.float32)
  dot_t = functools.partial(
      lax.dot_general, dimension_numbers=(((0,), (0,)), ((), ())),
      preferred_element_type=jnp.float32)
  oth = oemb_ref[...]
  emb = emb_ref[...]
  win = win_ref[...]
  rf = (win & 1).astype(jnp.float32)
  h = (dot(oth, w1a[...]) + dot(emb, w1b[...]) +
       rf * dot(emb - oth, w1ab[...]) +
       dot_t(eft_ref[...], w1c[...]) +
       dot_t(tft_ref[...], w1d[...]) + b1[...])
  h = jnp.maximum(h, 0.0)
  msg = dot(h, w2[...]) + b2[...]
  h2 = jnp.maximum(dot(emb, u1a[...]) + dot(msg, u1b[...]) + ub1[...], 0.0)
  upd = dot(h2, u2[...]) + ub2[...]
  valid = win >= 0
  nz = jnp.any(upd != 0.0, axis=1, keepdims=True)
  keep = valid & nz
  out_ref[...] = jnp.where(keep, upd, emb)


def _full2(i):
  return (0, 0)


_tc_mlp = pl.pallas_call(
    _tc_body,
    grid=(GRID,),
    in_specs=[
        pl.BlockSpec((BLK, 1), lambda i: (i, 0)),
        pl.BlockSpec((BLK, D), lambda i: (i, 0)),
        pl.BlockSpec((DE, BLK), lambda i: (0, i)),
        pl.BlockSpec((DE, BLK), lambda i: (0, i)),
        pl.BlockSpec((BLK, D), lambda i: (i, 0)),
        pl.BlockSpec((D, D), _full2),
        pl.BlockSpec((D, D), _full2),
        pl.BlockSpec((D, D), _full2),
        pl.BlockSpec((DE, D), _full2),
        pl.BlockSpec((DE, D), _full2),
        pl.BlockSpec((1, D), _full2),
        pl.BlockSpec((D, D), _full2),
        pl.BlockSpec((1, D), _full2),
        pl.BlockSpec((D, D), _full2),
        pl.BlockSpec((D, D), _full2),
        pl.BlockSpec((1, D), _full2),
        pl.BlockSpec((D, D), _full2),
        pl.BlockSpec((1, D), _full2),
    ],
    out_specs=pl.BlockSpec((BLK, D), lambda i: (i, 0)),
    out_shape=jax.ShapeDtypeStruct((NP, D), jnp.float32),
)


def kernel(node_embeddings, src_node_ids, dst_node_ids, edge_features,
           time_features, W1, b1, W2, b2, U1, ub1, U2, ub2):
  sid = src_node_ids.astype(jnp.int32)
  did = dst_node_ids.astype(jnp.int32)
  ef1 = edge_features.T.reshape(-1)
  tf1 = time_features.T.reshape(-1)

  partials = _sc_scatter_max(sid, did)
  winner, oemb, eft, tft = _sc_gather(
      partials, node_embeddings, sid, did, ef1, tf1)

  w1a, w1b = W1[:D], W1[D:2 * D]
  embp = jnp.pad(node_embeddings, ((0, NP - N_NODES), (0, 0)))
  out = _tc_mlp(
      winner.reshape(NP, 1), oemb, eft, tft, embp,
      w1a, w1b, w1a - w1b, W1[2 * D:2 * D + DE], W1[2 * D + DE:],
      b1.reshape(1, D), W2, b2.reshape(1, D),
      U1[:D], U1[D:], ub1.reshape(1, D), U2, ub2.reshape(1, D))
  return out[:N_NODES]
